# baseline (speedup 1.0000x reference)
import functools

import jax
import jax.numpy as jnp
from jax import lax
from jax.experimental import pallas as pl
from jax.experimental.pallas import tpu as pltpu
from jax.experimental.pallas import tpu_sc as plsc

BATCH = 16384
HIDDEN = 64
LANES = 16
N_CHUNKS = 4
CHUNK = 128
B_PER_W = N_CHUNKS * CHUNK
N_WORKERS = BATCH // B_PER_W
H_CHUNKS = HIDDEN // LANES


def _distmult_sc_body(head_idx_hbm, rel_idx_hbm, tail_idx_hbm, node_hbm,
                      rel_hbm, out_hbm, idx_h, idx_r, idx_t, h_rows, r_rows,
                      t_rows, pv, sem0, sem1, sem2, sem3):
    sems = (sem0, sem1, sem2, sem3)
    wid = lax.axis_index("s") * 2 + lax.axis_index("c")

    pltpu.sync_copy(head_idx_hbm.at[wid], idx_h)
    pltpu.sync_copy(rel_idx_hbm.at[wid], idx_r)
    pltpu.sync_copy(tail_idx_hbm.at[wid], idx_t)

    copies = []
    for j in range(N_CHUNKS):
        copies.append(
            pltpu.async_copy(node_hbm.at[idx_h.at[j]], h_rows.at[j], sems[j]))
        copies.append(
            pltpu.async_copy(node_hbm.at[idx_t.at[j]], t_rows.at[j], sems[j]))
        copies.append(
            pltpu.async_copy(rel_hbm.at[idx_r.at[j]], r_rows.at[j], sems[j]))

    for j in range(N_CHUNKS):
        copies[3 * j].wait()
        copies[3 * j + 1].wait()
        copies[3 * j + 2].wait()

        def row_body(row, _, j=j):
            acc = (h_rows[j, row, pl.ds(0, LANES)]
                   * r_rows[j, row, pl.ds(0, LANES)]
                   * t_rows[j, row, pl.ds(0, LANES)])
            for c in range(1, H_CHUNKS):
                acc = acc + (h_rows[j, row, pl.ds(c * LANES, LANES)]
                             * r_rows[j, row, pl.ds(c * LANES, LANES)]
                             * t_rows[j, row, pl.ds(c * LANES, LANES)])
            pv[j * CHUNK + row, :] = acc
            return 0

        lax.fori_loop(0, CHUNK, row_body, 0, unroll=8)

    pltpu.sync_copy(pv, out_hbm.at[pl.ds(wid * B_PER_W, B_PER_W)])


def _reduce_tc_body(pv_ref, out_ref):
    out_ref[0, 0, :] = jnp.sum(pv_ref[...], axis=-1)


def kernel(head_index, rel_type, tail_index, node_emb, rel_emb):
    head3d = head_index.reshape(N_WORKERS, N_CHUNKS, CHUNK)
    rel3d = rel_type.reshape(N_WORKERS, N_CHUNKS, CHUNK)
    tail3d = tail_index.reshape(N_WORKERS, N_CHUNKS, CHUNK)

    mesh = plsc.VectorSubcoreMesh(core_axis_name="c", subcore_axis_name="s")
    sc_run = functools.partial(
        pl.kernel,
        mesh=mesh,
        compiler_params=pltpu.CompilerParams(use_tc_tiling_on_sc=False),
        out_type=jax.ShapeDtypeStruct((BATCH, LANES), jnp.float32),
        scratch_types=[
            pltpu.VMEM((N_CHUNKS, CHUNK), jnp.int32),
            pltpu.VMEM((N_CHUNKS, CHUNK), jnp.int32),
            pltpu.VMEM((N_CHUNKS, CHUNK), jnp.int32),
            pltpu.VMEM((N_CHUNKS, CHUNK, HIDDEN), jnp.float32),
            pltpu.VMEM((N_CHUNKS, CHUNK, HIDDEN), jnp.float32),
            pltpu.VMEM((N_CHUNKS, CHUNK, HIDDEN), jnp.float32),
            pltpu.VMEM((B_PER_W, LANES), jnp.float32),
            pltpu.SemaphoreType.DMA,
            pltpu.SemaphoreType.DMA,
            pltpu.SemaphoreType.DMA,
            pltpu.SemaphoreType.DMA,
        ],
    )(_distmult_sc_body)
    partials = sc_run(head3d, rel3d, tail3d, node_emb, rel_emb)

    rows_per_blk = 512
    n_blk = BATCH // rows_per_blk
    out3 = pl.pallas_call(
        _reduce_tc_body,
        grid=(n_blk,),
        in_specs=[pl.BlockSpec((rows_per_blk, LANES), lambda i: (i, 0))],
        out_specs=pl.BlockSpec((1, 1, rows_per_blk), lambda i: (i, 0, 0)),
        out_shape=jax.ShapeDtypeStruct((n_blk, 1, rows_per_blk), jnp.float32),
    )(partials)
    return out3.reshape(BATCH)


# speedup vs baseline: 1.0043x; 1.0043x over previous
import functools

import jax
import jax.numpy as jnp
from jax import lax
from jax.experimental import pallas as pl
from jax.experimental.pallas import tpu as pltpu
from jax.experimental.pallas import tpu_sc as plsc

BATCH = 16384
HIDDEN = 64
LANES = 16
N_CHUNKS = 8
CHUNK = 64
B_PER_W = N_CHUNKS * CHUNK
N_WORKERS = BATCH // B_PER_W
H_CHUNKS = HIDDEN // LANES
ROWS_PER_LINE = 128 // LANES
PV_LINES = B_PER_W // ROWS_PER_LINE


def _distmult_sc_body(head_idx_hbm, rel_idx_hbm, tail_idx_hbm, off_hbm,
                      node_hbm, rel_hbm, out_hbm, idx_h, idx_r, idx_t, offs,
                      h_rows, r_rows, t_rows, pv, sem0, sem1):
    sems = (sem0, sem1)
    wid = lax.axis_index("s") * 2 + lax.axis_index("c")

    pltpu.sync_copy(head_idx_hbm.at[wid], idx_h)
    pltpu.sync_copy(rel_idx_hbm.at[wid], idx_r)
    pltpu.sync_copy(tail_idx_hbm.at[wid], idx_t)
    pltpu.sync_copy(off_hbm.at[wid], offs)

    def fire(j):
        ring = j % 2
        return (
            pltpu.async_copy(node_hbm.at[idx_h.at[j]], h_rows.at[ring],
                             sems[ring]),
            pltpu.async_copy(node_hbm.at[idx_t.at[j]], t_rows.at[ring],
                             sems[ring]),
            pltpu.async_copy(rel_hbm.at[idx_r.at[j]], r_rows.at[ring],
                             sems[ring]),
        )

    inflight = fire(0)
    for j in range(N_CHUNKS):
        for cp in inflight:
            cp.wait()
        if j + 1 < N_CHUNKS:
            nxt = fire(j + 1)
        ring = j % 2

        def row_body(row, _, j=j, ring=ring):
            ov = offs[j, row, :]
            oh = ov[0]
            orr = ov[1]
            ot = ov[2]
            acc = (h_rows[ring, row, pl.ds(oh, LANES)]
                   * r_rows[ring, row, pl.ds(orr, LANES)]
                   * t_rows[ring, row, pl.ds(ot, LANES)])
            for c in range(1, H_CHUNKS):
                acc = acc + (h_rows[ring, row, pl.ds(oh + c * LANES, LANES)]
                             * r_rows[ring, row, pl.ds(orr + c * LANES, LANES)]
                             * t_rows[ring, row, pl.ds(ot + c * LANES, LANES)])
            b = j * CHUNK + row
            pv[b // ROWS_PER_LINE, pl.ds((b % ROWS_PER_LINE) * LANES, LANES)] = acc
            return 0

        lax.fori_loop(0, CHUNK, row_body, 0, unroll=8)
        if j + 1 < N_CHUNKS:
            inflight = nxt

    pltpu.sync_copy(pv, out_hbm.at[pl.ds(wid * PV_LINES, PV_LINES)])


def _reduce_tc_body(pv_ref, out_ref):
    l_ids = lax.broadcasted_iota(jnp.int32, (128, ROWS_PER_LINE), 0)
    g_ids = lax.broadcasted_iota(jnp.int32, (128, ROWS_PER_LINE), 1)
    seg = jnp.where(l_ids // LANES == g_ids, 1.0, 0.0).astype(jnp.float32)
    out_ref[...] = jax.lax.dot_general(
        pv_ref[...], seg, (((1,), (0,)), ((), ())),
        preferred_element_type=jnp.float32)


def kernel(head_index, rel_type, tail_index, node_emb, rel_emb):
    head3d = (head_index // 2).reshape(N_WORKERS, N_CHUNKS, CHUNK)
    rel3d = (rel_type // 2).reshape(N_WORKERS, N_CHUNKS, CHUNK)
    tail3d = (tail_index // 2).reshape(N_WORKERS, N_CHUNKS, CHUNK)
    offs = jnp.stack([
        (head_index % 2) * HIDDEN,
        (rel_type % 2) * HIDDEN,
        (tail_index % 2) * HIDDEN,
    ], axis=-1)
    offs = jnp.pad(offs, ((0, 0), (0, LANES - 3)))
    offs = offs.reshape(N_WORKERS, N_CHUNKS, CHUNK, LANES)
    node2 = node_emb.reshape(node_emb.shape[0] // 2, 2 * HIDDEN)
    rel2 = rel_emb.reshape(rel_emb.shape[0] // 2, 2 * HIDDEN)

    mesh = plsc.VectorSubcoreMesh(core_axis_name="c", subcore_axis_name="s")
    sc_run = functools.partial(
        pl.kernel,
        mesh=mesh,
        out_type=jax.ShapeDtypeStruct((N_WORKERS * PV_LINES, 128),
                                      jnp.float32),
        scratch_types=[
            pltpu.VMEM((N_CHUNKS, CHUNK), jnp.int32),
            pltpu.VMEM((N_CHUNKS, CHUNK), jnp.int32),
            pltpu.VMEM((N_CHUNKS, CHUNK), jnp.int32),
            pltpu.VMEM((N_CHUNKS, CHUNK, LANES), jnp.int32),
            pltpu.VMEM((2, CHUNK, 2 * HIDDEN), jnp.float32),
            pltpu.VMEM((2, CHUNK, 2 * HIDDEN), jnp.float32),
            pltpu.VMEM((2, CHUNK, 2 * HIDDEN), jnp.float32),
            pltpu.VMEM((PV_LINES, 128), jnp.float32),
            pltpu.SemaphoreType.DMA,
            pltpu.SemaphoreType.DMA,
        ],
    )(_distmult_sc_body)
    partials = sc_run(head3d, rel3d, tail3d, offs, node2, rel2)

    rows_per_blk = 512
    n_blk = N_WORKERS * PV_LINES // rows_per_blk
    out2 = pl.pallas_call(
        _reduce_tc_body,
        grid=(n_blk,),
        in_specs=[pl.BlockSpec((rows_per_blk, 128), lambda i: (i, 0))],
        out_specs=pl.BlockSpec((rows_per_blk, ROWS_PER_LINE),
                               lambda i: (i, 0)),
        out_shape=jax.ShapeDtypeStruct((N_WORKERS * PV_LINES, ROWS_PER_LINE),
                                       jnp.float32),
    )(partials)
    return out2.reshape(BATCH)


# speedup vs baseline: 1.4975x; 1.4911x over previous
import functools

import jax
import jax.numpy as jnp
from jax import lax
from jax.experimental import pallas as pl
from jax.experimental.pallas import tpu as pltpu
from jax.experimental.pallas import tpu_sc as plsc

BATCH = 16384
HIDDEN = 64
LANES = 16
N_CHUNKS = 4
CHUNK = 128
B_PER_W = N_CHUNKS * CHUNK
N_WORKERS = BATCH // B_PER_W
NODE_BLK = 4096
NODE_SPLIT = 123 * NODE_BLK
REL_SPLIT = 256
RED_BLK = 2048


def _pack_body(a_ref, dst_ref):
    emb = a_ref[...].T
    dst_ref[:, :HIDDEN] = emb
    dst_ref[:, HIDDEN:] = emb


def _gather_table(table_t, blk):
    n = table_t.shape[1]
    nblk = pl.cdiv(n, blk)
    return pl.pallas_call(
        _pack_body,
        grid=(nblk,),
        in_specs=[pl.BlockSpec((HIDDEN, blk), lambda i: (0, i))],
        out_specs=pl.BlockSpec((blk, 2 * HIDDEN), lambda i: (i, 0)),
        out_shape=jax.ShapeDtypeStruct((nblk * blk, 2 * HIDDEN),
                                       jnp.float32),
    )(table_t)


def _reduce_body(p_ref, o_ref):
    o_ref[...] = jnp.sum(p_ref[...], axis=1)


def _distmult_sc_body(node_hbm, rel_hbm, head_idx_hbm, rel_idx_hbm,
                      tail_idx_hbm, out_hbm, idx_h, idx_r, idx_t, h_v, r_v,
                      t_v, pacc_v, sem_h, sem_r, sem_t):
    wid = lax.axis_index("s") * 2 + lax.axis_index("c")

    pltpu.sync_copy(head_idx_hbm.at[wid], idx_h)
    pltpu.sync_copy(rel_idx_hbm.at[wid], idx_r)
    pltpu.sync_copy(tail_idx_hbm.at[wid], idx_t)

    def gather(j, ring):
        return (
            pltpu.make_async_copy(node_hbm.at[idx_h.at[j]], h_v.at[ring],
                                  sem_h),
            pltpu.make_async_copy(rel_hbm.at[idx_r.at[j]], r_v.at[ring],
                                  sem_r),
            pltpu.make_async_copy(node_hbm.at[idx_t.at[j]], t_v.at[ring],
                                  sem_t),
        )

    for c in gather(0, 0):
        c.start()
    for j in range(N_CHUNKS):
        ring = j % 2
        for c in gather(j, ring):
            c.wait()
        if j + 1 < N_CHUNKS:
            for c in gather(j + 1, 1 - ring):
                c.start()

        def row_body(i, _, ring=ring):
            acc = jnp.zeros((LANES,), jnp.float32)
            for g in range(HIDDEN // LANES):
                sl = pl.ds(g * LANES, LANES)
                acc = acc + (h_v[ring, i, sl] * r_v[ring, i, sl]
                             * t_v[ring, i, sl])
            pacc_v[i] = acc
            return 0

        lax.fori_loop(0, CHUNK, row_body, 0)
        pltpu.sync_copy(
            pacc_v, out_hbm.at[pl.ds(wid * B_PER_W + j * CHUNK, CHUNK)])


def kernel(head_index, rel_type, tail_index, node_emb, rel_emb):
    head3d = head_index.reshape(N_WORKERS, N_CHUNKS, CHUNK)
    rel3d = rel_type.reshape(N_WORKERS, N_CHUNKS, CHUNK)
    tail3d = tail_index.reshape(N_WORKERS, N_CHUNKS, CHUNK)
    node_tab = _gather_table(node_emb.T, NODE_BLK)
    rel_tab = _gather_table(rel_emb.T, 512)

    mesh = plsc.VectorSubcoreMesh(core_axis_name="c", subcore_axis_name="s")
    sc_run = functools.partial(
        pl.kernel,
        mesh=mesh,
        compiler_params=pltpu.CompilerParams(use_tc_tiling_on_sc=True),
        out_type=jax.ShapeDtypeStruct((BATCH, LANES), jnp.float32),
        scratch_types=[
            pltpu.VMEM((N_CHUNKS, CHUNK), jnp.int32),
            pltpu.VMEM((N_CHUNKS, CHUNK), jnp.int32),
            pltpu.VMEM((N_CHUNKS, CHUNK), jnp.int32),
            pltpu.VMEM((2, CHUNK, 2 * HIDDEN), jnp.float32),
            pltpu.VMEM((2, CHUNK, 2 * HIDDEN), jnp.float32),
            pltpu.VMEM((2, CHUNK, 2 * HIDDEN), jnp.float32),
            pltpu.VMEM((CHUNK, LANES), jnp.float32),
            pltpu.SemaphoreType.DMA,
            pltpu.SemaphoreType.DMA,
            pltpu.SemaphoreType.DMA,
        ],
    )(_distmult_sc_body)
    pacc = sc_run(node_tab, rel_tab, head3d, rel3d, tail3d)

    return pl.pallas_call(
        _reduce_body,
        grid=(BATCH // RED_BLK,),
        in_specs=[pl.BlockSpec((RED_BLK, LANES), lambda i: (i, 0))],
        out_specs=pl.BlockSpec((RED_BLK,), lambda i: (i,)),
        out_shape=jax.ShapeDtypeStruct((BATCH,), jnp.float32),
    )(pacc)
